# Initial kernel scaffold; baseline (speedup 1.0000x reference)
import jax
import jax.numpy as jnp
from jax.experimental import pallas as pl


def kernel(locations, box_cls, box_regression, centerness, image_sizes):
    raise NotImplementedError("write your pallas kernel here")


# speedup vs baseline: 1.9513x; 1.9513x over previous
import jax
import jax.numpy as jnp
from jax.experimental import pallas as pl
from jax.experimental.pallas import tpu as pltpu

_PRE_NMS_THRESH = 0.05
_PRE_NMS_TOP_N = 1000
_NMS_THRESH = 0.6
_FPN_POST_NMS_TOP_N = 100
_STRIDE = 8.0
_M_PAD = 1024


def _score_kernel(cls_ref, cent_ref, out_ref):
    cls = cls_ref[0]
    cent = cent_ref[0]
    c_dim = cls.shape[-1]
    s = jax.nn.sigmoid(cls)
    lane = jax.lax.broadcasted_iota(jnp.int32, s.shape, 1)
    s = jnp.where(lane == c_dim - 1, 0.0, s)
    cand = s > _PRE_NMS_THRESH
    val = s * jax.nn.sigmoid(cent)
    out_ref[0] = jnp.where(cand, val, -1.0)


def _nms_kernel(bb_ref, bbt_ref, valid_ref, keep_ref, iou_ref):
    bb = bb_ref[0]
    bbt = bbt_ref[0]
    x1c, y1c, x2c, y2c = bb[:, 0:1], bb[:, 1:2], bb[:, 2:3], bb[:, 3:4]
    x1r, y1r, x2r, y2r = bbt[0:1, :], bbt[1:2, :], bbt[2:3, :], bbt[3:4, :]
    wx = jnp.clip(jnp.minimum(x2c, x2r) - jnp.maximum(x1c, x1r), 0.0, None)
    wy = jnp.clip(jnp.minimum(y2c, y2r) - jnp.maximum(y1c, y1r), 0.0, None)
    inter = wx * wy
    area_c = (x2c - x1c) * (y2c - y1c)
    area_r = (x2r - x1r) * (y2r - y1r)
    iou_ref[...] = inter / jnp.clip(area_c + area_r - inter, 1e-9, None)

    keep0 = valid_ref[0]
    m = keep0.shape[-1]
    lane = jax.lax.broadcasted_iota(jnp.int32, (1, m), 1)

    def body(i, keep):
        row = iou_ref[pl.ds(i, 1), :]
        keep_i = jnp.sum(jnp.where(lane == i, keep, 0.0)) > 0.0
        sup = (row > _NMS_THRESH) & (lane > i)
        return jnp.where(keep_i, jnp.where(sup, 0.0, keep), keep)

    keep_ref[0] = jax.lax.fori_loop(0, _PRE_NMS_TOP_N, body, keep0)


def kernel(locations, box_cls, box_regression, centerness, image_sizes):
    N, C, H, W = box_cls.shape
    HW = H * W

    cls_t = jnp.transpose(box_cls, (0, 2, 3, 1)).reshape(N, HW, C)
    cent_t = jnp.transpose(centerness, (0, 2, 3, 1)).reshape(N, HW, 1)

    blk = 2048
    masked = pl.pallas_call(
        _score_kernel,
        grid=(N, HW // blk),
        in_specs=[
            pl.BlockSpec((1, blk, C), lambda n, h: (n, h, 0)),
            pl.BlockSpec((1, blk, 1), lambda n, h: (n, h, 0)),
        ],
        out_specs=pl.BlockSpec((1, blk, C), lambda n, h: (n, h, 0)),
        out_shape=jax.ShapeDtypeStruct((N, HW, C), jnp.float32),
    )(cls_t, cent_t)

    flat = masked.reshape(N, HW * C)
    topv, topi = jax.lax.top_k(flat, _PRE_NMS_TOP_N)
    loc_idx = topi // C
    labels = topi % C + 1
    valid = topv > 0.0

    reg = jnp.transpose(box_regression, (0, 2, 3, 1)).reshape(N, HW, 4) * _STRIDE
    per_loc = locations[loc_idx]
    per_reg = jnp.take_along_axis(reg, loc_idx[..., None], axis=1)
    x1 = per_loc[..., 0] - per_reg[..., 0]
    y1 = per_loc[..., 1] - per_reg[..., 1]
    x2 = per_loc[..., 0] + per_reg[..., 2]
    y2 = per_loc[..., 1] + per_reg[..., 3]
    h_img = image_sizes[:, 0].astype(jnp.float32)[:, None]
    w_img = image_sizes[:, 1].astype(jnp.float32)[:, None]
    x1 = jnp.clip(x1, 0.0, w_img - 1.0)
    y1 = jnp.clip(y1, 0.0, h_img - 1.0)
    x2 = jnp.clip(x2, 0.0, w_img - 1.0)
    y2 = jnp.clip(y2, 0.0, h_img - 1.0)
    boxes = jnp.stack([x1, y1, x2, y2], axis=-1)
    scores = jnp.where(valid, jnp.sqrt(jnp.where(valid, topv, 1.0)), 0.0)

    max_coord = jnp.max(boxes)
    offs = labels.astype(jnp.float32) * (max_coord + 1.0)
    nms_boxes = boxes + offs[..., None]

    k = _PRE_NMS_TOP_N
    pad = _M_PAD - k
    bb = jnp.pad(nms_boxes, ((0, 0), (0, pad), (0, 0)))
    bbt = jnp.transpose(bb, (0, 2, 1))
    vpad = jnp.pad(valid.astype(jnp.float32), ((0, 0), (0, pad)))
    vrow = vpad[:, None, :]

    keep = pl.pallas_call(
        _nms_kernel,
        grid=(N,),
        in_specs=[
            pl.BlockSpec((1, _M_PAD, 4), lambda n: (n, 0, 0)),
            pl.BlockSpec((1, 4, _M_PAD), lambda n: (n, 0, 0)),
            pl.BlockSpec((1, 1, _M_PAD), lambda n: (n, 0, 0)),
        ],
        out_specs=pl.BlockSpec((1, 1, _M_PAD), lambda n: (n, 0, 0)),
        out_shape=jax.ShapeDtypeStruct((N, 1, _M_PAD), jnp.float32),
        scratch_shapes=[pltpu.VMEM((_M_PAD, _M_PAD), jnp.float32)],
    )(bb, bbt, vrow)

    keep = keep[:, 0, :k] > 0.0
    kept_scores = jnp.where(keep, scores, -1.0)
    fv, fi = jax.lax.top_k(kept_scores, _FPN_POST_NMS_TOP_N)
    out_valid = fv > 0.0
    out_boxes = jnp.take_along_axis(boxes, fi[..., None], axis=1)
    out_scores = jnp.take_along_axis(scores, fi, axis=1)
    out_labels = jnp.take_along_axis(labels, fi, axis=1)
    out_boxes = jnp.where(out_valid[..., None], out_boxes, 0.0)
    out_scores = jnp.where(out_valid, out_scores, 0.0)
    out_labels = jnp.where(out_valid, out_labels, 0)
    return out_boxes, out_scores, out_labels, out_valid
